# Initial kernel scaffold; baseline (speedup 1.0000x reference)
import jax
import jax.numpy as jnp
from jax.experimental import pallas as pl


def kernel(x, edge_index, cluster_assignment, W_self, W_neigh, b):
    raise NotImplementedError("write your pallas kernel here")


# speedup vs baseline: 5.4764x; 5.4764x over previous
import functools

import jax
import jax.numpy as jnp
from jax import lax
from jax.experimental import pallas as pl
from jax.experimental.pallas import tpu as pltpu
from jax.experimental.pallas import tpu_sc as plsc

N = 10000
E = 320000
D = 128
C = 1000

NC = 2
NS = 16
NW = NC * NS
CH = 80
EPW = E // NW
ECH = EPW // CH
NCHN = N // CH
STRIDE = 624
ZR = 640
C0 = 640
C1 = C - STRIDE

_MESH = plsc.VectorSubcoreMesh(core_axis_name="c", subcore_axis_name="s")


def _wid(cid, sid):
    return sid * NC + cid


@functools.partial(
    pl.kernel,
    mesh=_MESH,
    out_type=[
        jax.ShapeDtypeStruct((NC, N, D), jnp.float32),
        jax.ShapeDtypeStruct((NC, C, D), jnp.float32),
    ],
    scratch_types=[
        pltpu.VMEM((CH,), jnp.int32),
        pltpu.VMEM((CH, D), jnp.float32),
        pltpu.VMEM_SHARED((N, D), jnp.float32),
        pltpu.VMEM_SHARED((C, D), jnp.float32),
    ],
)
def _sc_counts(dst_flat, ca_flat, ones_hbm, zrows, degp, cntp, idx_c, ones_v,
               dacc, cacc):
    cid = lax.axis_index("c")
    sid = lax.axis_index("s")
    wid = _wid(cid, sid)
    row0 = sid * STRIDE
    pltpu.sync_copy(zrows, dacc.at[pl.ds(row0, ZR)])

    @pl.when(sid == 0)
    def _():
        pltpu.sync_copy(zrows.at[pl.ds(0, C0)], cacc.at[pl.ds(0, C0)])

    @pl.when(sid == 1)
    def _():
        pltpu.sync_copy(zrows.at[pl.ds(0, C1)], cacc.at[pl.ds(STRIDE, C1)])

    plsc.subcore_barrier()

    pltpu.sync_copy(ones_hbm, ones_v)
    ebase = wid * EPW

    def deg_body(g, carry):
        pltpu.sync_copy(dst_flat.at[pl.ds(ebase + g * CH, CH)], idx_c)
        pltpu.sync_copy(ones_v, dacc.at[idx_c], add=True)
        return carry

    lax.fori_loop(0, ECH, deg_body, 0)

    lo = wid * NCHN // NW
    hi = (wid + 1) * NCHN // NW

    def cnt_body(g, carry):
        pltpu.sync_copy(ca_flat.at[pl.ds(g * CH, CH)], idx_c)
        pltpu.sync_copy(ones_v, cacc.at[idx_c], add=True)
        return carry

    lax.fori_loop(lo, hi, cnt_body, 0)

    plsc.subcore_barrier()

    pltpu.sync_copy(dacc.at[pl.ds(row0, ZR)],
                    degp.at[cid].at[pl.ds(row0, ZR)])

    @pl.when(sid == 0)
    def _():
        pltpu.sync_copy(cacc.at[pl.ds(0, C0)], cntp.at[cid].at[pl.ds(0, C0)])

    @pl.when(sid == 1)
    def _():
        pltpu.sync_copy(cacc.at[pl.ds(STRIDE, C1)],
                        cntp.at[cid].at[pl.ds(STRIDE, C1)])


@functools.partial(
    pl.kernel,
    mesh=_MESH,
    out_type=jax.ShapeDtypeStruct((NC, N, D), jnp.float32),
    scratch_types=[
        pltpu.VMEM((ECH, CH), jnp.int32),
        pltpu.VMEM((CH,), jnp.int32),
        pltpu.VMEM((CH, D), jnp.float32),
        pltpu.VMEM_SHARED((N, D), jnp.float32),
        pltpu.SemaphoreType.DMA,
    ],
)
def _sc_aggregate(src3, dst_flat, h, zrows, aggp, src_v, dst_c, rows_v, acc,
                  sem):
    cid = lax.axis_index("c")
    sid = lax.axis_index("s")
    wid = _wid(cid, sid)
    row0 = sid * STRIDE
    pltpu.sync_copy(zrows, acc.at[pl.ds(row0, ZR)])
    plsc.subcore_barrier()

    pltpu.sync_copy(src3.at[wid], src_v)
    ebase = wid * EPW

    def body(g, carry):
        pltpu.sync_copy(dst_flat.at[pl.ds(ebase + g * CH, CH)], dst_c)
        pltpu.async_copy(h.at[src_v.at[g]], rows_v, sem).wait()
        pltpu.sync_copy(rows_v, acc.at[dst_c], add=True)
        return carry

    lax.fori_loop(0, ECH, body, 0)

    plsc.subcore_barrier()
    pltpu.sync_copy(acc.at[pl.ds(row0, ZR)],
                    aggp.at[cid].at[pl.ds(row0, ZR)])


@functools.partial(
    pl.kernel,
    mesh=_MESH,
    out_type=jax.ShapeDtypeStruct((NC, C, D), jnp.float32),
    scratch_types=[
        pltpu.VMEM((CH,), jnp.int32),
        pltpu.VMEM((CH, D), jnp.float32),
        pltpu.VMEM_SHARED((C, D), jnp.float32),
    ],
)
def _sc_cluster_sum(h, ca_flat, zrows, csump, ca_c, rows_v, cacc):
    cid = lax.axis_index("c")
    sid = lax.axis_index("s")
    wid = _wid(cid, sid)

    @pl.when(sid == 0)
    def _():
        pltpu.sync_copy(zrows.at[pl.ds(0, C0)], cacc.at[pl.ds(0, C0)])

    @pl.when(sid == 1)
    def _():
        pltpu.sync_copy(zrows.at[pl.ds(0, C1)], cacc.at[pl.ds(STRIDE, C1)])

    plsc.subcore_barrier()

    lo = wid * NCHN // NW
    hi = (wid + 1) * NCHN // NW

    def body(g, carry):
        pltpu.sync_copy(ca_flat.at[pl.ds(g * CH, CH)], ca_c)
        pltpu.sync_copy(h.at[pl.ds(g * CH, CH)], rows_v)
        pltpu.sync_copy(rows_v, cacc.at[ca_c], add=True)
        return carry

    lax.fori_loop(lo, hi, body, 0)

    plsc.subcore_barrier()

    @pl.when(sid == 0)
    def _():
        pltpu.sync_copy(cacc.at[pl.ds(0, C0)], csump.at[cid].at[pl.ds(0, C0)])

    @pl.when(sid == 1)
    def _():
        pltpu.sync_copy(cacc.at[pl.ds(STRIDE, C1)],
                        csump.at[cid].at[pl.ds(STRIDE, C1)])


@functools.partial(
    pl.kernel,
    mesh=_MESH,
    out_type=jax.ShapeDtypeStruct((N, D), jnp.float32),
    scratch_types=[
        pltpu.VMEM((CH,), jnp.int32),
        pltpu.VMEM((CH, D), jnp.float32),
        pltpu.SemaphoreType.DMA,
    ],
)
def _sc_gather(cmean, ca_flat, out, ca_c, rows_v, sem):
    cid = lax.axis_index("c")
    sid = lax.axis_index("s")
    wid = _wid(cid, sid)
    lo = wid * NCHN // NW
    hi = (wid + 1) * NCHN // NW

    def body(g, carry):
        pltpu.sync_copy(ca_flat.at[pl.ds(g * CH, CH)], ca_c)
        pltpu.async_copy(cmean.at[ca_c], rows_v, sem).wait()
        pltpu.sync_copy(rows_v, out.at[pl.ds(g * CH, CH)])
        return carry

    lax.fori_loop(lo, hi, body, 0)


BN = 1000


def _tc_layer_body(relu, h_ref, aggp_ref, degp_ref, ws_ref, wn_ref, b_ref,
                   out_ref):
    deg = degp_ref[0, :, 0:1] + degp_ref[1, :, 0:1]
    deg = jnp.maximum(deg, 1.0)
    agg = (aggp_ref[0] + aggp_ref[1]) / deg
    acc = jnp.dot(h_ref[...], ws_ref[...], preferred_element_type=jnp.float32)
    acc += jnp.dot(agg, wn_ref[...], preferred_element_type=jnp.float32)
    acc += b_ref[...]
    if relu:
        acc = jnp.maximum(acc, 0.0)
    out_ref[...] = acc


def _tc_layer(h, aggp, degp, ws, wn, b2, relu):
    grid = (N // BN,)
    return pl.pallas_call(
        functools.partial(_tc_layer_body, relu),
        grid=grid,
        in_specs=[
            pl.BlockSpec((BN, D), lambda i: (i, 0)),
            pl.BlockSpec((NC, BN, D), lambda i: (0, i, 0)),
            pl.BlockSpec((NC, BN, D), lambda i: (0, i, 0)),
            pl.BlockSpec((D, D), lambda i: (0, 0)),
            pl.BlockSpec((D, D), lambda i: (0, 0)),
            pl.BlockSpec((1, D), lambda i: (0, 0)),
        ],
        out_specs=pl.BlockSpec((BN, D), lambda i: (i, 0)),
        out_shape=jax.ShapeDtypeStruct((N, D), jnp.float32),
    )(h, aggp, degp, ws, wn, b2)


def _tc_cmean_body(csump_ref, cntp_ref, out_ref):
    cnt = cntp_ref[0, :, 0:1] + cntp_ref[1, :, 0:1]
    cnt = jnp.maximum(cnt, 1.0)
    out_ref[...] = (csump_ref[0] + csump_ref[1]) / cnt


def _tc_cmean(csump, cntp):
    return pl.pallas_call(
        _tc_cmean_body,
        in_specs=[
            pl.BlockSpec((NC, C, D), lambda: (0, 0, 0)),
            pl.BlockSpec((NC, C, D), lambda: (0, 0, 0)),
        ],
        out_specs=pl.BlockSpec((C, D), lambda: (0, 0)),
        out_shape=jax.ShapeDtypeStruct((C, D), jnp.float32),
    )(csump, cntp)


def kernel(x, edge_index, cluster_assignment, W_self, W_neigh, b):
    src3 = edge_index[0].reshape(NW, ECH, CH).astype(jnp.int32)
    dst_flat = edge_index[1].astype(jnp.int32)
    ca_flat = cluster_assignment.astype(jnp.int32)
    zrows = jnp.zeros((ZR, D), jnp.float32)
    ones_rows = jnp.ones((CH, D), jnp.float32)

    degp, cntp = _sc_counts(dst_flat, ca_flat, ones_rows, zrows)

    h = x
    for i in range(3):
        aggp = _sc_aggregate(src3, dst_flat, h, zrows)
        h = _tc_layer(h, aggp, degp, W_self[i], W_neigh[i],
                      b[i].reshape(1, D), relu=(i < 2))

    csump = _sc_cluster_sum(h, ca_flat, zrows)
    cmean = _tc_cmean(csump, cntp)
    return _sc_gather(cmean, ca_flat)


# speedup vs baseline: 8.9637x; 1.6368x over previous
import functools

import jax
import jax.numpy as jnp
from jax import lax
from jax.experimental import pallas as pl
from jax.experimental.pallas import tpu as pltpu
from jax.experimental.pallas import tpu_sc as plsc

N = 10000
E = 320000
D = 128
C = 1000

NC = 2
NS = 16
NW = NC * NS
CH = 80
EPW = E // NW
ECH = EPW // CH
NCHN = N // CH
STRIDE = 624
ZR = 640
C0 = 640
C1 = C - STRIDE

_MESH = plsc.VectorSubcoreMesh(core_axis_name="c", subcore_axis_name="s")


def _wid(cid, sid):
    return sid * NC + cid


@functools.partial(
    pl.kernel,
    mesh=_MESH,
    out_type=[
        jax.ShapeDtypeStruct((NC, N, D), jnp.float32),
        jax.ShapeDtypeStruct((NC, C, D), jnp.float32),
    ],
    scratch_types=[
        pltpu.VMEM((CH,), jnp.int32),
        pltpu.VMEM((CH,), jnp.int32),
        pltpu.VMEM((CH, D), jnp.float32),
        pltpu.VMEM_SHARED((N, D), jnp.float32),
        pltpu.VMEM_SHARED((C, D), jnp.float32),
        pltpu.SemaphoreType.DMA,
        pltpu.SemaphoreType.DMA,
    ],
)
def _sc_counts(dst_flat, ca_flat, ones_hbm, zrows, degp, cntp, idx_c0,
               idx_c1, ones_v, dacc, cacc, sem0, sem1):
    cid = lax.axis_index("c")
    sid = lax.axis_index("s")
    wid = _wid(cid, sid)
    row0 = sid * STRIDE
    pltpu.sync_copy(zrows, dacc.at[pl.ds(row0, ZR)])

    @pl.when(sid == 0)
    def _():
        pltpu.sync_copy(zrows.at[pl.ds(0, C0)], cacc.at[pl.ds(0, C0)])

    @pl.when(sid == 1)
    def _():
        pltpu.sync_copy(zrows.at[pl.ds(0, C1)], cacc.at[pl.ds(STRIDE, C1)])

    plsc.subcore_barrier()

    pltpu.sync_copy(ones_hbm, ones_v)
    ebase = wid * EPW

    pltpu.sync_copy(dst_flat.at[pl.ds(ebase, CH)], idx_c0)
    pltpu.async_copy(ones_v, dacc.at[idx_c0], sem0, add=True)

    def deg_body(k, carry):
        a = 2 * k
        pltpu.sync_copy(dst_flat.at[pl.ds(ebase + (a + 1) * CH, CH)], idx_c1)
        pltpu.async_copy(ones_v, dacc.at[idx_c1], sem1, add=True)
        pltpu.make_async_copy(ones_v, dacc.at[idx_c0], sem0).wait()
        pltpu.sync_copy(dst_flat.at[pl.ds(ebase + (a + 2) * CH, CH)], idx_c0)
        pltpu.async_copy(ones_v, dacc.at[idx_c0], sem0, add=True)
        pltpu.make_async_copy(ones_v, dacc.at[idx_c1], sem1).wait()
        return carry

    lax.fori_loop(0, (ECH - 1) // 2, deg_body, 0)
    pltpu.make_async_copy(ones_v, dacc.at[idx_c0], sem0).wait()

    lo = wid * NCHN // NW
    hi = (wid + 1) * NCHN // NW

    def cnt_body(g, carry):
        pltpu.sync_copy(ca_flat.at[pl.ds(g * CH, CH)], idx_c0)
        pltpu.sync_copy(ones_v, cacc.at[idx_c0], add=True)
        return carry

    lax.fori_loop(lo, hi, cnt_body, 0)

    plsc.subcore_barrier()

    pltpu.sync_copy(dacc.at[pl.ds(row0, ZR)],
                    degp.at[cid].at[pl.ds(row0, ZR)])

    @pl.when(sid == 0)
    def _():
        pltpu.sync_copy(cacc.at[pl.ds(0, C0)], cntp.at[cid].at[pl.ds(0, C0)])

    @pl.when(sid == 1)
    def _():
        pltpu.sync_copy(cacc.at[pl.ds(STRIDE, C1)],
                        cntp.at[cid].at[pl.ds(STRIDE, C1)])


@functools.partial(
    pl.kernel,
    mesh=_MESH,
    out_type=jax.ShapeDtypeStruct((NC, N, D), jnp.float32),
    scratch_types=[
        pltpu.VMEM((ECH, CH), jnp.int32),
        pltpu.VMEM((CH,), jnp.int32),
        pltpu.VMEM((CH,), jnp.int32),
        pltpu.VMEM((CH, D), jnp.float32),
        pltpu.VMEM((CH, D), jnp.float32),
        pltpu.VMEM_SHARED((N, D), jnp.float32),
        pltpu.SemaphoreType.DMA,
        pltpu.SemaphoreType.DMA,
    ],
)
def _sc_aggregate(src3, dst_flat, h, zrows, aggp, src_v, dst_c0, dst_c1,
                  rows0, rows1, acc, sem0, sem1):
    cid = lax.axis_index("c")
    sid = lax.axis_index("s")
    wid = _wid(cid, sid)
    row0 = sid * STRIDE
    pltpu.sync_copy(zrows, acc.at[pl.ds(row0, ZR)])
    plsc.subcore_barrier()

    pltpu.sync_copy(src3.at[wid], src_v)
    ebase = wid * EPW

    pltpu.sync_copy(dst_flat.at[pl.ds(ebase, CH)], dst_c0)
    pltpu.async_copy(h.at[src_v.at[0]], rows0, sem0)

    def body(k, carry):
        a = 2 * k
        pltpu.sync_copy(dst_flat.at[pl.ds(ebase + (a + 1) * CH, CH)], dst_c1)
        pltpu.async_copy(h.at[src_v.at[a + 1]], rows1, sem1)
        pltpu.make_async_copy(h.at[src_v.at[a]], rows0, sem0).wait()
        pltpu.sync_copy(rows0, acc.at[dst_c0], add=True)
        pltpu.sync_copy(dst_flat.at[pl.ds(ebase + (a + 2) * CH, CH)], dst_c0)
        pltpu.async_copy(h.at[src_v.at[a + 2]], rows0, sem0)
        pltpu.make_async_copy(h.at[src_v.at[a + 1]], rows1, sem1).wait()
        pltpu.sync_copy(rows1, acc.at[dst_c1], add=True)
        return carry

    lax.fori_loop(0, (ECH - 1) // 2, body, 0)

    pltpu.make_async_copy(h.at[src_v.at[ECH - 1]], rows0, sem0).wait()
    pltpu.sync_copy(rows0, acc.at[dst_c0], add=True)

    plsc.subcore_barrier()
    pltpu.sync_copy(acc.at[pl.ds(row0, ZR)],
                    aggp.at[cid].at[pl.ds(row0, ZR)])


@functools.partial(
    pl.kernel,
    mesh=_MESH,
    out_type=jax.ShapeDtypeStruct((NC, C, D), jnp.float32),
    scratch_types=[
        pltpu.VMEM((CH,), jnp.int32),
        pltpu.VMEM((CH, D), jnp.float32),
        pltpu.VMEM_SHARED((C, D), jnp.float32),
    ],
)
def _sc_cluster_sum(h, ca_flat, zrows, csump, ca_c, rows_v, cacc):
    cid = lax.axis_index("c")
    sid = lax.axis_index("s")
    wid = _wid(cid, sid)

    @pl.when(sid == 0)
    def _():
        pltpu.sync_copy(zrows.at[pl.ds(0, C0)], cacc.at[pl.ds(0, C0)])

    @pl.when(sid == 1)
    def _():
        pltpu.sync_copy(zrows.at[pl.ds(0, C1)], cacc.at[pl.ds(STRIDE, C1)])

    plsc.subcore_barrier()

    lo = wid * NCHN // NW
    hi = (wid + 1) * NCHN // NW

    def body(g, carry):
        pltpu.sync_copy(ca_flat.at[pl.ds(g * CH, CH)], ca_c)
        pltpu.sync_copy(h.at[pl.ds(g * CH, CH)], rows_v)
        pltpu.sync_copy(rows_v, cacc.at[ca_c], add=True)
        return carry

    lax.fori_loop(lo, hi, body, 0)

    plsc.subcore_barrier()

    @pl.when(sid == 0)
    def _():
        pltpu.sync_copy(cacc.at[pl.ds(0, C0)], csump.at[cid].at[pl.ds(0, C0)])

    @pl.when(sid == 1)
    def _():
        pltpu.sync_copy(cacc.at[pl.ds(STRIDE, C1)],
                        csump.at[cid].at[pl.ds(STRIDE, C1)])


@functools.partial(
    pl.kernel,
    mesh=_MESH,
    out_type=jax.ShapeDtypeStruct((N, D), jnp.float32),
    scratch_types=[
        pltpu.VMEM((CH,), jnp.int32),
        pltpu.VMEM((CH, D), jnp.float32),
        pltpu.SemaphoreType.DMA,
    ],
)
def _sc_gather(cmean, ca_flat, out, ca_c, rows_v, sem):
    cid = lax.axis_index("c")
    sid = lax.axis_index("s")
    wid = _wid(cid, sid)
    lo = wid * NCHN // NW
    hi = (wid + 1) * NCHN // NW

    def body(g, carry):
        pltpu.sync_copy(ca_flat.at[pl.ds(g * CH, CH)], ca_c)
        pltpu.async_copy(cmean.at[ca_c], rows_v, sem).wait()
        pltpu.sync_copy(rows_v, out.at[pl.ds(g * CH, CH)])
        return carry

    lax.fori_loop(lo, hi, body, 0)


BN = 1000


def _tc_layer_body(relu, h_ref, aggp_ref, degp_ref, ws_ref, wn_ref, b_ref,
                   out_ref):
    deg = degp_ref[0, :, 0:1] + degp_ref[1, :, 0:1]
    deg = jnp.maximum(deg, 1.0)
    agg = (aggp_ref[0] + aggp_ref[1]) / deg
    acc = jnp.dot(h_ref[...], ws_ref[...], preferred_element_type=jnp.float32)
    acc += jnp.dot(agg, wn_ref[...], preferred_element_type=jnp.float32)
    acc += b_ref[...]
    if relu:
        acc = jnp.maximum(acc, 0.0)
    out_ref[...] = acc


def _tc_layer(h, aggp, degp, ws, wn, b2, relu):
    grid = (N // BN,)
    return pl.pallas_call(
        functools.partial(_tc_layer_body, relu),
        grid=grid,
        in_specs=[
            pl.BlockSpec((BN, D), lambda i: (i, 0)),
            pl.BlockSpec((NC, BN, D), lambda i: (0, i, 0)),
            pl.BlockSpec((NC, BN, D), lambda i: (0, i, 0)),
            pl.BlockSpec((D, D), lambda i: (0, 0)),
            pl.BlockSpec((D, D), lambda i: (0, 0)),
            pl.BlockSpec((1, D), lambda i: (0, 0)),
        ],
        out_specs=pl.BlockSpec((BN, D), lambda i: (i, 0)),
        out_shape=jax.ShapeDtypeStruct((N, D), jnp.float32),
    )(h, aggp, degp, ws, wn, b2)


def _tc_cmean_body(csump_ref, cntp_ref, out_ref):
    cnt = cntp_ref[0, :, 0:1] + cntp_ref[1, :, 0:1]
    cnt = jnp.maximum(cnt, 1.0)
    out_ref[...] = (csump_ref[0] + csump_ref[1]) / cnt


def _tc_cmean(csump, cntp):
    return pl.pallas_call(
        _tc_cmean_body,
        in_specs=[
            pl.BlockSpec((NC, C, D), lambda: (0, 0, 0)),
            pl.BlockSpec((NC, C, D), lambda: (0, 0, 0)),
        ],
        out_specs=pl.BlockSpec((C, D), lambda: (0, 0)),
        out_shape=jax.ShapeDtypeStruct((C, D), jnp.float32),
    )(csump, cntp)


def kernel(x, edge_index, cluster_assignment, W_self, W_neigh, b):
    src3 = edge_index[0].reshape(NW, ECH, CH).astype(jnp.int32)
    dst_flat = edge_index[1].astype(jnp.int32)
    ca_flat = cluster_assignment.astype(jnp.int32)
    zrows = jnp.zeros((ZR, D), jnp.float32)
    ones_rows = jnp.ones((CH, D), jnp.float32)

    degp, cntp = _sc_counts(dst_flat, ca_flat, ones_rows, zrows)

    h = x
    for i in range(3):
        aggp = _sc_aggregate(src3, dst_flat, h, zrows)
        h = _tc_layer(h, aggp, degp, W_self[i], W_neigh[i],
                      b[i].reshape(1, D), relu=(i < 2))

    csump = _sc_cluster_sum(h, ca_flat, zrows)
    cmean = _tc_cmean(csump, cntp)
    return _sc_gather(cmean, ca_flat)


# speedup vs baseline: 10.2374x; 1.1421x over previous
import functools

import jax
import jax.numpy as jnp
from jax import lax
from jax.experimental import pallas as pl
from jax.experimental.pallas import tpu as pltpu
from jax.experimental.pallas import tpu_sc as plsc

N = 10000
E = 320000
D = 128
C = 1000

NC = 2
NS = 16
NW = NC * NS
CH = 80
EPW = E // NW
ECH = EPW // CH
NCHN = N // CH
STRIDE = 624
ZR = 640
C0 = 640
C1 = C - STRIDE

_MESH = plsc.VectorSubcoreMesh(core_axis_name="c", subcore_axis_name="s")


def _wid(cid, sid):
    return sid * NC + cid


@functools.partial(
    pl.kernel,
    mesh=_MESH,
    out_type=[
        jax.ShapeDtypeStruct((NC, N, D), jnp.float32),
        jax.ShapeDtypeStruct((NC, C, D), jnp.float32),
    ],
    scratch_types=[
        pltpu.VMEM((CH,), jnp.int32),
        pltpu.VMEM((CH,), jnp.int32),
        pltpu.VMEM((CH, D), jnp.float32),
        pltpu.VMEM_SHARED((N, D), jnp.float32),
        pltpu.VMEM_SHARED((C, D), jnp.float32),
        pltpu.SemaphoreType.DMA,
        pltpu.SemaphoreType.DMA,
    ],
)
def _sc_counts(dst_flat, ca_flat, ones_hbm, zrows, degp, cntp, idx_c0,
               idx_c1, ones_v, dacc, cacc, sem0, sem1):
    cid = lax.axis_index("c")
    sid = lax.axis_index("s")
    wid = _wid(cid, sid)
    row0 = sid * STRIDE
    pltpu.sync_copy(zrows, dacc.at[pl.ds(row0, ZR)])

    @pl.when(sid == 0)
    def _():
        pltpu.sync_copy(zrows.at[pl.ds(0, C0)], cacc.at[pl.ds(0, C0)])

    @pl.when(sid == 1)
    def _():
        pltpu.sync_copy(zrows.at[pl.ds(0, C1)], cacc.at[pl.ds(STRIDE, C1)])

    plsc.subcore_barrier()

    pltpu.sync_copy(ones_hbm, ones_v)
    ebase = wid * EPW

    pltpu.sync_copy(dst_flat.at[pl.ds(ebase, CH)], idx_c0)
    pltpu.async_copy(ones_v, dacc.at[idx_c0], sem0, add=True)

    def deg_body(k, carry):
        a = 2 * k
        pltpu.sync_copy(dst_flat.at[pl.ds(ebase + (a + 1) * CH, CH)], idx_c1)
        pltpu.async_copy(ones_v, dacc.at[idx_c1], sem1, add=True)
        pltpu.make_async_copy(ones_v, dacc.at[idx_c0], sem0).wait()
        pltpu.sync_copy(dst_flat.at[pl.ds(ebase + (a + 2) * CH, CH)], idx_c0)
        pltpu.async_copy(ones_v, dacc.at[idx_c0], sem0, add=True)
        pltpu.make_async_copy(ones_v, dacc.at[idx_c1], sem1).wait()
        return carry

    lax.fori_loop(0, (ECH - 1) // 2, deg_body, 0)
    pltpu.make_async_copy(ones_v, dacc.at[idx_c0], sem0).wait()

    lo = wid * NCHN // NW
    hi = (wid + 1) * NCHN // NW

    def cnt_body(g, carry):
        pltpu.sync_copy(ca_flat.at[pl.ds(g * CH, CH)], idx_c0)
        pltpu.sync_copy(ones_v, cacc.at[idx_c0], add=True)
        return carry

    lax.fori_loop(lo, hi, cnt_body, 0)

    plsc.subcore_barrier()

    pltpu.sync_copy(dacc.at[pl.ds(row0, ZR)],
                    degp.at[cid].at[pl.ds(row0, ZR)])

    @pl.when(sid == 0)
    def _():
        pltpu.sync_copy(cacc.at[pl.ds(0, C0)], cntp.at[cid].at[pl.ds(0, C0)])

    @pl.when(sid == 1)
    def _():
        pltpu.sync_copy(cacc.at[pl.ds(STRIDE, C1)],
                        cntp.at[cid].at[pl.ds(STRIDE, C1)])


@functools.partial(
    pl.kernel,
    mesh=_MESH,
    out_type=jax.ShapeDtypeStruct((NC, N, D), jnp.float32),
    scratch_types=[
        pltpu.VMEM((ECH, CH), jnp.int32),
        pltpu.VMEM((CH,), jnp.int32),
        pltpu.VMEM((CH,), jnp.int32),
        pltpu.VMEM((CH, D), jnp.float32),
        pltpu.VMEM((CH, D), jnp.float32),
        pltpu.VMEM_SHARED((N, D), jnp.float32),
        pltpu.SemaphoreType.DMA,
        pltpu.SemaphoreType.DMA,
        pltpu.SemaphoreType.DMA,
        pltpu.SemaphoreType.DMA,
    ],
)
def _sc_aggregate(src3, dst_flat, h, zrows, aggp, src_v, dst_c0, dst_c1,
                  rows0, rows1, acc, sem0, sem1, isem0, isem1):
    cid = lax.axis_index("c")
    sid = lax.axis_index("s")
    wid = _wid(cid, sid)
    row0 = sid * STRIDE
    pltpu.sync_copy(zrows, acc.at[pl.ds(row0, ZR)])
    plsc.subcore_barrier()

    pltpu.sync_copy(src3.at[wid], src_v)
    ebase = wid * EPW

    def idx_start(a, c, s):
        return pltpu.async_copy(dst_flat.at[pl.ds(ebase + a * CH, CH)], c, s)

    def idx_wait(a, c, s):
        pltpu.make_async_copy(dst_flat.at[pl.ds(ebase + a * CH, CH)], c,
                              s).wait()

    def gat_start(a, r, s):
        return pltpu.async_copy(h.at[src_v.at[a]], r, s)

    def gat_wait(a, r, s):
        pltpu.make_async_copy(h.at[src_v.at[a]], r, s).wait()

    idx_start(0, dst_c0, isem0)
    idx_start(1, dst_c1, isem1)
    gat_start(0, rows0, sem0)
    gat_start(1, rows1, sem1)

    def body(k, carry):
        a = 2 * k
        gat_wait(a, rows0, sem0)
        idx_wait(a, dst_c0, isem0)
        pltpu.sync_copy(rows0, acc.at[dst_c0], add=True)
        idx_start(a + 2, dst_c0, isem0)
        gat_start(a + 2, rows0, sem0)
        gat_wait(a + 1, rows1, sem1)
        idx_wait(a + 1, dst_c1, isem1)
        pltpu.sync_copy(rows1, acc.at[dst_c1], add=True)

        @pl.when(a + 3 < ECH)
        def _():
            idx_start(a + 3, dst_c1, isem1)
            gat_start(a + 3, rows1, sem1)

        return carry

    lax.fori_loop(0, (ECH - 1) // 2, body, 0)

    gat_wait(ECH - 1, rows0, sem0)
    idx_wait(ECH - 1, dst_c0, isem0)
    pltpu.sync_copy(rows0, acc.at[dst_c0], add=True)

    plsc.subcore_barrier()
    pltpu.sync_copy(acc.at[pl.ds(row0, ZR)],
                    aggp.at[cid].at[pl.ds(row0, ZR)])


@functools.partial(
    pl.kernel,
    mesh=_MESH,
    out_type=jax.ShapeDtypeStruct((NC, C, D), jnp.float32),
    scratch_types=[
        pltpu.VMEM((CH,), jnp.int32),
        pltpu.VMEM((CH, D), jnp.float32),
        pltpu.VMEM_SHARED((C, D), jnp.float32),
    ],
)
def _sc_cluster_sum(h, ca_flat, zrows, csump, ca_c, rows_v, cacc):
    cid = lax.axis_index("c")
    sid = lax.axis_index("s")
    wid = _wid(cid, sid)

    @pl.when(sid == 0)
    def _():
        pltpu.sync_copy(zrows.at[pl.ds(0, C0)], cacc.at[pl.ds(0, C0)])

    @pl.when(sid == 1)
    def _():
        pltpu.sync_copy(zrows.at[pl.ds(0, C1)], cacc.at[pl.ds(STRIDE, C1)])

    plsc.subcore_barrier()

    lo = wid * NCHN // NW
    hi = (wid + 1) * NCHN // NW

    def body(g, carry):
        pltpu.sync_copy(ca_flat.at[pl.ds(g * CH, CH)], ca_c)
        pltpu.sync_copy(h.at[pl.ds(g * CH, CH)], rows_v)
        pltpu.sync_copy(rows_v, cacc.at[ca_c], add=True)
        return carry

    lax.fori_loop(lo, hi, body, 0)

    plsc.subcore_barrier()

    @pl.when(sid == 0)
    def _():
        pltpu.sync_copy(cacc.at[pl.ds(0, C0)], csump.at[cid].at[pl.ds(0, C0)])

    @pl.when(sid == 1)
    def _():
        pltpu.sync_copy(cacc.at[pl.ds(STRIDE, C1)],
                        csump.at[cid].at[pl.ds(STRIDE, C1)])


@functools.partial(
    pl.kernel,
    mesh=_MESH,
    out_type=jax.ShapeDtypeStruct((N, D), jnp.float32),
    scratch_types=[
        pltpu.VMEM((CH,), jnp.int32),
        pltpu.VMEM((CH, D), jnp.float32),
        pltpu.SemaphoreType.DMA,
    ],
)
def _sc_gather(cmean, ca_flat, out, ca_c, rows_v, sem):
    cid = lax.axis_index("c")
    sid = lax.axis_index("s")
    wid = _wid(cid, sid)
    lo = wid * NCHN // NW
    hi = (wid + 1) * NCHN // NW

    def body(g, carry):
        pltpu.sync_copy(ca_flat.at[pl.ds(g * CH, CH)], ca_c)
        pltpu.async_copy(cmean.at[ca_c], rows_v, sem).wait()
        pltpu.sync_copy(rows_v, out.at[pl.ds(g * CH, CH)])
        return carry

    lax.fori_loop(lo, hi, body, 0)


BN = 1000


def _tc_layer_body(relu, h_ref, aggp_ref, degp_ref, ws_ref, wn_ref, b_ref,
                   out_ref):
    deg = degp_ref[0, :, 0:1] + degp_ref[1, :, 0:1]
    deg = jnp.maximum(deg, 1.0)
    agg = (aggp_ref[0] + aggp_ref[1]) / deg
    acc = jnp.dot(h_ref[...], ws_ref[...], preferred_element_type=jnp.float32)
    acc += jnp.dot(agg, wn_ref[...], preferred_element_type=jnp.float32)
    acc += b_ref[...]
    if relu:
        acc = jnp.maximum(acc, 0.0)
    out_ref[...] = acc


def _tc_layer(h, aggp, degp, ws, wn, b2, relu):
    grid = (N // BN,)
    return pl.pallas_call(
        functools.partial(_tc_layer_body, relu),
        grid=grid,
        in_specs=[
            pl.BlockSpec((BN, D), lambda i: (i, 0)),
            pl.BlockSpec((NC, BN, D), lambda i: (0, i, 0)),
            pl.BlockSpec((NC, BN, D), lambda i: (0, i, 0)),
            pl.BlockSpec((D, D), lambda i: (0, 0)),
            pl.BlockSpec((D, D), lambda i: (0, 0)),
            pl.BlockSpec((1, D), lambda i: (0, 0)),
        ],
        out_specs=pl.BlockSpec((BN, D), lambda i: (i, 0)),
        out_shape=jax.ShapeDtypeStruct((N, D), jnp.float32),
    )(h, aggp, degp, ws, wn, b2)


def _tc_cmean_body(csump_ref, cntp_ref, out_ref):
    cnt = cntp_ref[0, :, 0:1] + cntp_ref[1, :, 0:1]
    cnt = jnp.maximum(cnt, 1.0)
    out_ref[...] = (csump_ref[0] + csump_ref[1]) / cnt


def _tc_cmean(csump, cntp):
    return pl.pallas_call(
        _tc_cmean_body,
        in_specs=[
            pl.BlockSpec((NC, C, D), lambda: (0, 0, 0)),
            pl.BlockSpec((NC, C, D), lambda: (0, 0, 0)),
        ],
        out_specs=pl.BlockSpec((C, D), lambda: (0, 0)),
        out_shape=jax.ShapeDtypeStruct((C, D), jnp.float32),
    )(csump, cntp)


def kernel(x, edge_index, cluster_assignment, W_self, W_neigh, b):
    src3 = edge_index[0].reshape(NW, ECH, CH).astype(jnp.int32)
    dst_flat = edge_index[1].astype(jnp.int32)
    ca_flat = cluster_assignment.astype(jnp.int32)
    zrows = jnp.zeros((ZR, D), jnp.float32)
    ones_rows = jnp.ones((CH, D), jnp.float32)

    degp, cntp = _sc_counts(dst_flat, ca_flat, ones_rows, zrows)

    h = x
    for i in range(3):
        aggp = _sc_aggregate(src3, dst_flat, h, zrows)
        h = _tc_layer(h, aggp, degp, W_self[i], W_neigh[i],
                      b[i].reshape(1, D), relu=(i < 2))

    csump = _sc_cluster_sum(h, ca_flat, zrows)
    cmean = _tc_cmean(csump, cntp)
    return _sc_gather(cmean, ca_flat)


# speedup vs baseline: 11.4084x; 1.1144x over previous
import functools

import jax
import jax.numpy as jnp
from jax import lax
from jax.experimental import pallas as pl
from jax.experimental.pallas import tpu as pltpu
from jax.experimental.pallas import tpu_sc as plsc

N = 10000
E = 320000
D = 128
C = 1000

NC = 2
NS = 16
NW = NC * NS
CH = 80
EPW = E // NW
ECH = EPW // CH
NCHN = N // CH
STRIDE = 624
ZR = 640
C0 = 640
C1 = C - STRIDE

_MESH = plsc.VectorSubcoreMesh(core_axis_name="c", subcore_axis_name="s")


def _wid(cid, sid):
    return sid * NC + cid


@functools.partial(
    pl.kernel,
    mesh=_MESH,
    out_type=[
        jax.ShapeDtypeStruct((NC, N, D), jnp.float32),
        jax.ShapeDtypeStruct((NC, C, D), jnp.float32),
    ],
    scratch_types=[
        pltpu.VMEM((CH,), jnp.int32),
        pltpu.VMEM((CH,), jnp.int32),
        pltpu.VMEM((CH, D), jnp.float32),
        pltpu.VMEM_SHARED((N, D), jnp.float32),
        pltpu.VMEM_SHARED((C, D), jnp.float32),
        pltpu.SemaphoreType.DMA,
        pltpu.SemaphoreType.DMA,
    ],
)
def _sc_counts(dst_flat, ca_flat, ones_hbm, zrows, degp, cntp, idx_c0,
               idx_c1, ones_v, dacc, cacc, sem0, sem1):
    cid = lax.axis_index("c")
    sid = lax.axis_index("s")
    wid = _wid(cid, sid)
    row0 = sid * STRIDE
    pltpu.sync_copy(zrows, dacc.at[pl.ds(row0, ZR)])

    @pl.when(sid == 0)
    def _():
        pltpu.sync_copy(zrows.at[pl.ds(0, C0)], cacc.at[pl.ds(0, C0)])

    @pl.when(sid == 1)
    def _():
        pltpu.sync_copy(zrows.at[pl.ds(0, C1)], cacc.at[pl.ds(STRIDE, C1)])

    plsc.subcore_barrier()

    pltpu.sync_copy(ones_hbm, ones_v)
    ebase = wid * EPW

    pltpu.sync_copy(dst_flat.at[pl.ds(ebase, CH)], idx_c0)
    pltpu.async_copy(ones_v, dacc.at[idx_c0], sem0, add=True)

    def deg_body(k, carry):
        a = 2 * k
        pltpu.sync_copy(dst_flat.at[pl.ds(ebase + (a + 1) * CH, CH)], idx_c1)
        pltpu.async_copy(ones_v, dacc.at[idx_c1], sem1, add=True)
        pltpu.make_async_copy(ones_v, dacc.at[idx_c0], sem0).wait()
        pltpu.sync_copy(dst_flat.at[pl.ds(ebase + (a + 2) * CH, CH)], idx_c0)
        pltpu.async_copy(ones_v, dacc.at[idx_c0], sem0, add=True)
        pltpu.make_async_copy(ones_v, dacc.at[idx_c1], sem1).wait()
        return carry

    lax.fori_loop(0, (ECH - 1) // 2, deg_body, 0)
    pltpu.make_async_copy(ones_v, dacc.at[idx_c0], sem0).wait()

    lo = wid * NCHN // NW
    hi = (wid + 1) * NCHN // NW

    def cnt_body(g, carry):
        pltpu.sync_copy(ca_flat.at[pl.ds(g * CH, CH)], idx_c0)
        pltpu.sync_copy(ones_v, cacc.at[idx_c0], add=True)
        return carry

    lax.fori_loop(lo, hi, cnt_body, 0)

    plsc.subcore_barrier()

    pltpu.sync_copy(dacc.at[pl.ds(row0, ZR)],
                    degp.at[cid].at[pl.ds(row0, ZR)])

    @pl.when(sid == 0)
    def _():
        pltpu.sync_copy(cacc.at[pl.ds(0, C0)], cntp.at[cid].at[pl.ds(0, C0)])

    @pl.when(sid == 1)
    def _():
        pltpu.sync_copy(cacc.at[pl.ds(STRIDE, C1)],
                        cntp.at[cid].at[pl.ds(STRIDE, C1)])


@functools.partial(
    pl.kernel,
    mesh=_MESH,
    out_type=jax.ShapeDtypeStruct((NC, N, D), jnp.float32),
    scratch_types=[
        pltpu.VMEM((ECH, CH), jnp.int32),
        pltpu.VMEM((CH,), jnp.int32),
        pltpu.VMEM((CH,), jnp.int32),
        pltpu.VMEM((CH,), jnp.int32),
        pltpu.VMEM((CH, D), jnp.float32),
        pltpu.VMEM((CH, D), jnp.float32),
        pltpu.VMEM((CH, D), jnp.float32),
        pltpu.VMEM_SHARED((N, D), jnp.float32),
        pltpu.SemaphoreType.DMA,
        pltpu.SemaphoreType.DMA,
        pltpu.SemaphoreType.DMA,
        pltpu.SemaphoreType.DMA,
        pltpu.SemaphoreType.DMA,
        pltpu.SemaphoreType.DMA,
        pltpu.SemaphoreType.DMA,
        pltpu.SemaphoreType.DMA,
        pltpu.SemaphoreType.DMA,
    ],
)
def _sc_aggregate(src3, dst_flat, h, zrows, aggp, src_v, dc0, dc1, dc2,
                  rw0, rw1, rw2, acc, gs0, gs1, gs2,
                  is0, is1, is2, ss0, ss1, ss2):
    dst_c = [dc0, dc1, dc2]
    rows = [rw0, rw1, rw2]
    gsem = [gs0, gs1, gs2]
    isem = [is0, is1, is2]
    ssem = [ss0, ss1, ss2]
    cid = lax.axis_index("c")
    sid = lax.axis_index("s")
    wid = _wid(cid, sid)
    row0 = sid * STRIDE
    pltpu.sync_copy(zrows, acc.at[pl.ds(row0, ZR)])
    plsc.subcore_barrier()

    pltpu.sync_copy(src3.at[wid], src_v)
    ebase = wid * EPW

    def idx_start(a, c, s):
        return pltpu.async_copy(dst_flat.at[pl.ds(ebase + a * CH, CH)], c, s)

    def idx_wait(a, c, s):
        pltpu.make_async_copy(dst_flat.at[pl.ds(ebase + a * CH, CH)], c,
                              s).wait()

    def gat_start(a, r, s):
        return pltpu.async_copy(h.at[src_v.at[a]], r, s)

    def gat_wait(a, r, s):
        pltpu.make_async_copy(h.at[src_v.at[a]], r, s).wait()

    def sca_start(a, b):
        return pltpu.async_copy(rows[b], acc.at[dst_c[b]], ssem[b], add=True)

    def sca_wait(b):
        pltpu.make_async_copy(rows[b], acc.at[dst_c[b]], ssem[b]).wait()

    def phase(a, b, prefetch, swait):
        gat_wait(a, rows[b], gsem[b])
        idx_wait(a, dst_c[b], isem[b])
        sca_start(a, b)
        b2 = (b + 2) % 3
        if swait:
            sca_wait(b2)
        if prefetch:
            idx_start(a + 2, dst_c[b2], isem[b2])
            gat_start(a + 2, rows[b2], gsem[b2])

    idx_start(0, dst_c[0], isem[0])
    gat_start(0, rows[0], gsem[0])
    idx_start(1, dst_c[1], isem[1])
    gat_start(1, rows[1], gsem[1])

    phase(0, 0, True, False)
    phase(1, 1, True, True)

    def body(k, carry):
        base = 3 * k + 2
        for p in range(3):
            phase(base + p, (2 + p) % 3, True, True)
        return carry

    lax.fori_loop(0, (ECH - 5) // 3, body, 0)

    phase(ECH - 3, 2, True, True)
    phase(ECH - 2, 0, False, True)
    phase(ECH - 1, 1, False, True)
    sca_wait(1)

    plsc.subcore_barrier()
    pltpu.sync_copy(acc.at[pl.ds(row0, ZR)],
                    aggp.at[cid].at[pl.ds(row0, ZR)])


@functools.partial(
    pl.kernel,
    mesh=_MESH,
    out_type=jax.ShapeDtypeStruct((NC, C, D), jnp.float32),
    scratch_types=[
        pltpu.VMEM((CH,), jnp.int32),
        pltpu.VMEM((CH, D), jnp.float32),
        pltpu.VMEM_SHARED((C, D), jnp.float32),
    ],
)
def _sc_cluster_sum(h, ca_flat, zrows, csump, ca_c, rows_v, cacc):
    cid = lax.axis_index("c")
    sid = lax.axis_index("s")
    wid = _wid(cid, sid)

    @pl.when(sid == 0)
    def _():
        pltpu.sync_copy(zrows.at[pl.ds(0, C0)], cacc.at[pl.ds(0, C0)])

    @pl.when(sid == 1)
    def _():
        pltpu.sync_copy(zrows.at[pl.ds(0, C1)], cacc.at[pl.ds(STRIDE, C1)])

    plsc.subcore_barrier()

    lo = wid * NCHN // NW
    hi = (wid + 1) * NCHN // NW

    def body(g, carry):
        pltpu.sync_copy(ca_flat.at[pl.ds(g * CH, CH)], ca_c)
        pltpu.sync_copy(h.at[pl.ds(g * CH, CH)], rows_v)
        pltpu.sync_copy(rows_v, cacc.at[ca_c], add=True)
        return carry

    lax.fori_loop(lo, hi, body, 0)

    plsc.subcore_barrier()

    @pl.when(sid == 0)
    def _():
        pltpu.sync_copy(cacc.at[pl.ds(0, C0)], csump.at[cid].at[pl.ds(0, C0)])

    @pl.when(sid == 1)
    def _():
        pltpu.sync_copy(cacc.at[pl.ds(STRIDE, C1)],
                        csump.at[cid].at[pl.ds(STRIDE, C1)])


@functools.partial(
    pl.kernel,
    mesh=_MESH,
    out_type=jax.ShapeDtypeStruct((N, D), jnp.float32),
    scratch_types=[
        pltpu.VMEM((CH,), jnp.int32),
        pltpu.VMEM((CH, D), jnp.float32),
        pltpu.SemaphoreType.DMA,
    ],
)
def _sc_gather(cmean, ca_flat, out, ca_c, rows_v, sem):
    cid = lax.axis_index("c")
    sid = lax.axis_index("s")
    wid = _wid(cid, sid)
    lo = wid * NCHN // NW
    hi = (wid + 1) * NCHN // NW

    def body(g, carry):
        pltpu.sync_copy(ca_flat.at[pl.ds(g * CH, CH)], ca_c)
        pltpu.async_copy(cmean.at[ca_c], rows_v, sem).wait()
        pltpu.sync_copy(rows_v, out.at[pl.ds(g * CH, CH)])
        return carry

    lax.fori_loop(lo, hi, body, 0)


BN = 1000


def _tc_layer_body(relu, h_ref, aggp_ref, degp_ref, ws_ref, wn_ref, b_ref,
                   out_ref):
    deg = degp_ref[0, :, 0:1] + degp_ref[1, :, 0:1]
    deg = jnp.maximum(deg, 1.0)
    agg = (aggp_ref[0] + aggp_ref[1]) / deg
    acc = jnp.dot(h_ref[...], ws_ref[...], preferred_element_type=jnp.float32)
    acc += jnp.dot(agg, wn_ref[...], preferred_element_type=jnp.float32)
    acc += b_ref[...]
    if relu:
        acc = jnp.maximum(acc, 0.0)
    out_ref[...] = acc


def _tc_layer(h, aggp, degp, ws, wn, b2, relu):
    grid = (N // BN,)
    return pl.pallas_call(
        functools.partial(_tc_layer_body, relu),
        grid=grid,
        in_specs=[
            pl.BlockSpec((BN, D), lambda i: (i, 0)),
            pl.BlockSpec((NC, BN, D), lambda i: (0, i, 0)),
            pl.BlockSpec((NC, BN, D), lambda i: (0, i, 0)),
            pl.BlockSpec((D, D), lambda i: (0, 0)),
            pl.BlockSpec((D, D), lambda i: (0, 0)),
            pl.BlockSpec((1, D), lambda i: (0, 0)),
        ],
        out_specs=pl.BlockSpec((BN, D), lambda i: (i, 0)),
        out_shape=jax.ShapeDtypeStruct((N, D), jnp.float32),
    )(h, aggp, degp, ws, wn, b2)


def _tc_cmean_body(csump_ref, cntp_ref, out_ref):
    cnt = cntp_ref[0, :, 0:1] + cntp_ref[1, :, 0:1]
    cnt = jnp.maximum(cnt, 1.0)
    out_ref[...] = (csump_ref[0] + csump_ref[1]) / cnt


def _tc_cmean(csump, cntp):
    return pl.pallas_call(
        _tc_cmean_body,
        in_specs=[
            pl.BlockSpec((NC, C, D), lambda: (0, 0, 0)),
            pl.BlockSpec((NC, C, D), lambda: (0, 0, 0)),
        ],
        out_specs=pl.BlockSpec((C, D), lambda: (0, 0)),
        out_shape=jax.ShapeDtypeStruct((C, D), jnp.float32),
    )(csump, cntp)


def kernel(x, edge_index, cluster_assignment, W_self, W_neigh, b):
    src3 = edge_index[0].reshape(NW, ECH, CH).astype(jnp.int32)
    dst_flat = edge_index[1].astype(jnp.int32)
    ca_flat = cluster_assignment.astype(jnp.int32)
    zrows = jnp.zeros((ZR, D), jnp.float32)
    ones_rows = jnp.ones((CH, D), jnp.float32)

    degp, cntp = _sc_counts(dst_flat, ca_flat, ones_rows, zrows)

    h = x
    for i in range(3):
        aggp = _sc_aggregate(src3, dst_flat, h, zrows)
        h = _tc_layer(h, aggp, degp, W_self[i], W_neigh[i],
                      b[i].reshape(1, D), relu=(i < 2))

    csump = _sc_cluster_sum(h, ca_flat, zrows)
    cmean = _tc_cmean(csump, cntp)
    return _sc_gather(cmean, ca_flat)
